# baseline (speedup 1.0000x reference)
import functools

import jax
import jax.numpy as jnp
from jax import lax
from jax.experimental import pallas as pl
from jax.experimental.pallas import tpu as pltpu
from jax.experimental.pallas import tpu_sc as plsc

N = 10000
E = 160000
D = 256
H = 128
NC = 2
NS = 16
K = 80
NCHUNK = 125
NP = 10240
RPTS = NP // NS


def _sc_body(x_hbm, edge_hbm, zrows_hbm, zcnt_hbm, ones_hbm,
             sums_hbm, counts_hbm,
             ibuf, rows, onesv, ssum, scnt,
             semg0, semg1, sems0, sems1, semc0, semc1):
    c = lax.axis_index("c")
    s = lax.axis_index("s")
    col0 = c * H
    semg = (semg0, semg1)
    sems = (sems0, sems1)
    semc = (semc0, semc1)

    pltpu.sync_copy(zrows_hbm, ssum.at[pl.ds(s * RPTS, RPTS)])
    pltpu.sync_copy(zcnt_hbm, scnt.at[pl.ds(s * RPTS, RPTS)])
    pltpu.sync_copy(ones_hbm, onesv)
    plsc.subcore_barrier()

    def load_idx(k, buf):
        pltpu.sync_copy(edge_hbm.at[0, s, k], ibuf.at[buf, 0])
        pltpu.sync_copy(edge_hbm.at[1, s, k], ibuf.at[buf, 1])

    def gather(buf):
        pltpu.async_copy(x_hbm.at[ibuf.at[buf, 0], pl.ds(col0, H)],
                         rows.at[buf], semg[buf])

    def wait_gather(buf):
        pltpu.make_async_copy(x_hbm.at[ibuf.at[buf, 0], pl.ds(col0, H)],
                              rows.at[buf], semg[buf]).wait()

    def scatter(buf):
        pltpu.async_copy(rows.at[buf], ssum.at[ibuf.at[buf, 1]],
                         sems[buf], add=True)
        pltpu.async_copy(onesv, scnt.at[ibuf.at[buf, 1]],
                         semc[buf], add=True)

    def wait_scatter(buf):
        pltpu.make_async_copy(rows.at[buf], ssum.at[ibuf.at[buf, 1]],
                              sems[buf]).wait()
        pltpu.make_async_copy(onesv, scnt.at[ibuf.at[buf, 1]],
                              semc[buf]).wait()

    def first_half(k0):
        load_idx(k0 + 1, 1)
        gather(1)
        wait_gather(0)
        scatter(0)

    def second_half(k0):
        wait_scatter(0)
        load_idx(k0 + 2, 0)
        gather(0)
        wait_gather(1)
        scatter(1)

    load_idx(0, 0)
    gather(0)
    first_half(0)
    second_half(0)

    def pair(i, carry):
        k0 = 2 * i
        wait_scatter(1)
        first_half(k0)
        second_half(k0)
        return carry

    lax.fori_loop(1, (NCHUNK - 1) // 2, pair, 0)
    wait_scatter(1)
    wait_gather(0)
    scatter(0)
    wait_scatter(0)

    plsc.subcore_barrier()

    pltpu.sync_copy(ssum.at[pl.ds(s * RPTS, RPTS)],
                    sums_hbm.at[pl.ds(c * NP + s * RPTS, RPTS)])
    pltpu.sync_copy(scnt.at[pl.ds(s * RPTS, RPTS)],
                    counts_hbm.at[pl.ds(c * NP + s * RPTS, RPTS)])


def _sc_aggregate(x, edge_rs, zrows, zcnt, ones):
    mesh = plsc.VectorSubcoreMesh(core_axis_name="c", subcore_axis_name="s")
    return pl.kernel(
        _sc_body,
        out_type=[
            jax.ShapeDtypeStruct((NC * NP, H), jnp.float32),
            jax.ShapeDtypeStruct((NC * NP,), jnp.float32),
        ],
        mesh=mesh,
        scratch_types=[
            pltpu.VMEM((2, 2, K), jnp.int32),
            pltpu.VMEM((2, K, H), jnp.float32),
            pltpu.VMEM((K,), jnp.float32),
            pltpu.VMEM_SHARED((NP, H), jnp.float32),
            pltpu.VMEM_SHARED((NP,), jnp.float32),
            pltpu.SemaphoreType.DMA,
            pltpu.SemaphoreType.DMA,
            pltpu.SemaphoreType.DMA,
            pltpu.SemaphoreType.DMA,
            pltpu.SemaphoreType.DMA,
            pltpu.SemaphoreType.DMA,
        ],
    )(x, edge_rs, zrows, zcnt, ones)


def _tc_body(x_ref, sums_ref, cnt_ref, w1t_ref, w2at_ref, w2bt_ref,
             b_ref, gamma_ref, beta_ref, out_ref):
    x = x_ref[...]
    s0 = sums_ref[pl.ds(0, N), :]
    s1 = sums_ref[pl.ds(NP, N), :]
    rec = 1.0 / jnp.maximum(cnt_ref[...], 1.0)
    m = jnp.dot(x, w1t_ref[...], preferred_element_type=jnp.float32)
    agg = (jnp.dot(s0, w2at_ref[...], preferred_element_type=jnp.float32)
           + jnp.dot(s1, w2bt_ref[...], preferred_element_type=jnp.float32))
    m = m + agg * rec + b_ref[...]
    mean = jnp.mean(m, axis=0, keepdims=True)
    d = m - mean
    var = jnp.mean(d * d, axis=0, keepdims=True)
    y = d * lax.rsqrt(var + 1e-5) * gamma_ref[...] + beta_ref[...]
    out_ref[...] = 0.5 * y * (1.0 + lax.erf(y * 0.7071067811865475))


def _tc_fused(x, sums_all, cnt, w1t, w2at, w2bt, b2, gamma2, beta2):
    return pl.pallas_call(
        _tc_body,
        out_shape=jax.ShapeDtypeStruct((N, D), jnp.float32),
    )(x, sums_all, cnt, w1t, w2at, w2bt, b2, gamma2, beta2)


@jax.jit
def kernel(x, edge_index, W, b, gamma, beta):
    src = edge_index[0]
    dst = edge_index[1]

    edge_rs = edge_index.reshape(2, NS, NCHUNK, K)
    zrows = jnp.zeros((RPTS, H), jnp.float32)
    zcnt = jnp.zeros((RPTS,), jnp.float32)
    ones = jnp.ones((K,), jnp.float32)

    sums_all, counts_all = _sc_aggregate(x, edge_rs, zrows, zcnt, ones)

    cnt = counts_all[:N][:, None]
    w1t = W[:, :D].T
    w2at = W[:, D:D + H].T
    w2bt = W[:, D + H:].T
    return _tc_fused(x, sums_all, cnt, w1t, w2at, w2bt,
                     b[None, :], gamma[None, :], beta[None, :])


# speedup vs baseline: 1.1990x; 1.1990x over previous
import functools

import jax
import jax.numpy as jnp
from jax import lax
from jax.experimental import pallas as pl
from jax.experimental.pallas import tpu as pltpu
from jax.experimental.pallas import tpu_sc as plsc

N = 10000
E = 160000
D = 256
H = 128
NC = 2
NS = 16
K = 80
NCHUNK = 125
NP = 10240
RPTS = NP // NS


def _sc_body(xsp_hbm, edge_hbm, zrows_hbm, zcnt_hbm, ones_hbm,
             sums_hbm, counts_hbm,
             sbuf, dbuf, rows, onesv, ssum, scnt,
             semg0, semg1, sems0, sems1, semc0, semc1,
             semis0, semis1, semid0, semid1):
    c = lax.axis_index("c")
    s = lax.axis_index("s")
    semg = (semg0, semg1)
    sems = (sems0, sems1)
    semc = (semc0, semc1)
    semis = (semis0, semis1)
    semid = (semid0, semid1)
    xtab = xsp_hbm.at[c]

    pltpu.sync_copy(zrows_hbm, ssum.at[pl.ds(s * RPTS, RPTS)])
    pltpu.sync_copy(zcnt_hbm, scnt.at[pl.ds(s * RPTS, RPTS)])
    pltpu.sync_copy(ones_hbm, onesv)
    plsc.subcore_barrier()

    def load_sidx(k, a):
        kc = jnp.minimum(k, NCHUNK - 1)
        pltpu.async_copy(edge_hbm.at[0, s, kc], sbuf.at[a], semis[a])

    def wait_sidx(a):
        pltpu.make_async_copy(edge_hbm.at[0, s, 0], sbuf.at[a],
                              semis[a]).wait()

    def load_didx(k, a):
        kc = jnp.minimum(k, NCHUNK - 1)
        pltpu.async_copy(edge_hbm.at[1, s, kc], dbuf.at[a], semid[a])

    def wait_didx(a):
        pltpu.make_async_copy(edge_hbm.at[1, s, 0], dbuf.at[a],
                              semid[a]).wait()

    def gather(a):
        pltpu.async_copy(xtab.at[sbuf.at[a]], rows.at[a], semg[a])

    def wait_gather(a):
        pltpu.make_async_copy(xtab.at[sbuf.at[a]], rows.at[a],
                              semg[a]).wait()

    def scatter(a):
        pltpu.async_copy(rows.at[a], ssum.at[dbuf.at[a]], sems[a], add=True)
        pltpu.async_copy(onesv, scnt.at[dbuf.at[a]], semc[a], add=True)

    def wait_scatter(a):
        pltpu.make_async_copy(rows.at[a], ssum.at[dbuf.at[a]],
                              sems[a]).wait()
        pltpu.make_async_copy(onesv, scnt.at[dbuf.at[a]], semc[a]).wait()

    def body(j, a, skip_wait_scatter=False):
        b = 1 - a
        wait_gather(a)
        load_sidx(j + 2, a)
        if not skip_wait_scatter:
            wait_scatter(b)
        load_didx(j + 1, b)
        wait_didx(a)
        scatter(a)
        wait_sidx(b)
        gather(b)

    load_sidx(0, 0)
    load_didx(0, 0)
    wait_sidx(0)
    gather(0)
    load_sidx(1, 1)
    body(0, 0, skip_wait_scatter=True)

    def pair(i, carry):
        body(2 * i + 1, 1)
        body(2 * i + 2, 0)
        return carry

    lax.fori_loop(0, (NCHUNK - 1) // 2, pair, 0)
    wait_sidx(0)
    wait_didx(1)
    wait_gather(1)
    wait_scatter(0)

    plsc.subcore_barrier()

    pltpu.sync_copy(ssum.at[pl.ds(s * RPTS, RPTS)],
                    sums_hbm.at[pl.ds(c * NP + s * RPTS, RPTS)])
    pltpu.sync_copy(scnt.at[pl.ds(s * RPTS, RPTS)],
                    counts_hbm.at[pl.ds(c * NP + s * RPTS, RPTS)])


def _sc_aggregate(xsp, edge_rs, zrows, zcnt, ones):
    mesh = plsc.VectorSubcoreMesh(core_axis_name="c", subcore_axis_name="s")
    return pl.kernel(
        _sc_body,
        out_type=[
            jax.ShapeDtypeStruct((NC * NP, H), jnp.float32),
            jax.ShapeDtypeStruct((NC * NP,), jnp.float32),
        ],
        mesh=mesh,
        scratch_types=[
            pltpu.VMEM((2, K), jnp.int32),
            pltpu.VMEM((2, K), jnp.int32),
            pltpu.VMEM((2, K, H), jnp.float32),
            pltpu.VMEM((K,), jnp.float32),
            pltpu.VMEM_SHARED((NP, H), jnp.float32),
            pltpu.VMEM_SHARED((NP,), jnp.float32),
        ] + [pltpu.SemaphoreType.DMA] * 10,
    )(xsp, edge_rs, zrows, zcnt, ones)


def _tc_body(x_ref, sums_ref, cnt_ref, w1t_ref, w2at_ref, w2bt_ref,
             b_ref, gamma_ref, beta_ref, out_ref):
    x = x_ref[...]
    s0 = sums_ref[pl.ds(0, N), :]
    s1 = sums_ref[pl.ds(NP, N), :]
    rec = 1.0 / jnp.maximum(cnt_ref[...], 1.0)
    m = jnp.dot(x, w1t_ref[...], preferred_element_type=jnp.float32)
    agg = (jnp.dot(s0, w2at_ref[...], preferred_element_type=jnp.float32)
           + jnp.dot(s1, w2bt_ref[...], preferred_element_type=jnp.float32))
    m = m + agg * rec + b_ref[...]
    mean = jnp.mean(m, axis=0, keepdims=True)
    d = m - mean
    var = jnp.mean(d * d, axis=0, keepdims=True)
    y = d * lax.rsqrt(var + 1e-5) * gamma_ref[...] + beta_ref[...]
    out_ref[...] = 0.5 * y * (1.0 + lax.erf(y * 0.7071067811865475))


def _tc_fused(x, sums_all, cnt, w1t, w2at, w2bt, b2, gamma2, beta2):
    return pl.pallas_call(
        _tc_body,
        out_shape=jax.ShapeDtypeStruct((N, D), jnp.float32),
    )(x, sums_all, cnt, w1t, w2at, w2bt, b2, gamma2, beta2)


@jax.jit
def kernel(x, edge_index, W, b, gamma, beta):
    src = edge_index[0]
    dst = edge_index[1]

    edge_rs = edge_index.reshape(2, NS, NCHUNK, K)
    xsp = x.reshape(N, 2, H).transpose(1, 0, 2)
    zrows = jnp.zeros((RPTS, H), jnp.float32)
    zcnt = jnp.zeros((RPTS,), jnp.float32)
    ones = jnp.ones((K,), jnp.float32)

    sums_all, counts_all = _sc_aggregate(xsp, edge_rs, zrows, zcnt, ones)

    cnt = counts_all[:N][:, None]
    w1t = W[:, :D].T
    w2at = W[:, D:D + H].T
    w2bt = W[:, D + H:].T
    return _tc_fused(x, sums_all, cnt, w1t, w2at, w2bt,
                     b[None, :], gamma[None, :], beta[None, :])
